# Initial kernel scaffold; baseline (speedup 1.0000x reference)
import jax
import jax.numpy as jnp
from jax.experimental import pallas as pl


def kernel(batch, node_types, feat_map, ip_feats, domain_feats, url_feats, W_ip, b_ip, W_dom, b_dom, W_url, b_url):
    raise NotImplementedError("write your pallas kernel here")


# speedup vs baseline: 2.6712x; 2.6712x over previous
import functools

import jax
import jax.numpy as jnp
from jax import lax
from jax.experimental import pallas as pl
from jax.experimental.pallas import tpu as pltpu
from jax.experimental.pallas import tpu_sc as plsc

OUT_DIM = 64
NUM_NODE_TYPES = 5
FEAT = 128
B = 16384
NUM_CORES = 2
NUM_SUBCORES = 16
NW = NUM_CORES * NUM_SUBCORES
BPW = B // NW


def _sc_gather_body(batch_hbm, nt_hbm, fm_hbm, t0_hbm, t1_hbm, t2_hbm,
                    ty_out, g0_out, g1_out, g2_out,
                    bidx_v, ty_v, fi_v, rows_v, sem):
  wid = lax.axis_index("s") * NUM_CORES + lax.axis_index("c")
  base = wid * BPW
  pltpu.sync_copy(batch_hbm.at[pl.ds(base, BPW)], bidx_v)
  pltpu.async_copy(nt_hbm.at[bidx_v], ty_v, sem).wait()
  pltpu.sync_copy(ty_v, ty_out.at[pl.ds(base, BPW)])
  pltpu.async_copy(fm_hbm.at[bidx_v], fi_v, sem).wait()
  for tbl, out in ((t0_hbm, g0_out), (t1_hbm, g1_out), (t2_hbm, g2_out)):
    pltpu.async_copy(tbl.at[fi_v], rows_v, sem).wait()
    pltpu.sync_copy(rows_v, out.at[pl.ds(base, BPW)])


@functools.cache
def _sc_gather():
  return pl.kernel(
      _sc_gather_body,
      out_type=[
          jax.ShapeDtypeStruct((B,), jnp.int32),
          jax.ShapeDtypeStruct((B, FEAT), jnp.float32),
          jax.ShapeDtypeStruct((B, FEAT), jnp.float32),
          jax.ShapeDtypeStruct((B, FEAT), jnp.float32),
      ],
      mesh=plsc.VectorSubcoreMesh(core_axis_name="c", subcore_axis_name="s"),
      scratch_types=[
          pltpu.VMEM((BPW,), jnp.int32),
          pltpu.VMEM((BPW,), jnp.int32),
          pltpu.VMEM((BPW,), jnp.int32),
          pltpu.VMEM((BPW, FEAT), jnp.float32),
          pltpu.SemaphoreType.DMA,
      ],
  )


def _tc_project_body(ty_ref, g0_ref, g1_ref, g2_ref, w_ref, b_ref, o_ref):
  ty = ty_ref[...]
  cols = lax.broadcasted_iota(jnp.int32, (BPW, OUT_DIM), 1)
  acc = (cols == ty).astype(jnp.float32)
  for t, g_ref in enumerate((g0_ref, g1_ref, g2_ref)):
    p = jnp.dot(g_ref[...], w_ref[t], preferred_element_type=jnp.float32,
                precision=lax.Precision.HIGHEST)
    p = p + b_ref[t][None, :]
    acc = acc + jnp.where(ty == t, p, 0.0)
  o_ref[...] = acc


def _tc_project(ty3, g0, g1, g2, wp, bp):
  return pl.pallas_call(
      _tc_project_body,
      grid=(NW,),
      in_specs=[
          pl.BlockSpec((BPW, 1), lambda i: (i, 0)),
          pl.BlockSpec((BPW, FEAT), lambda i: (i, 0)),
          pl.BlockSpec((BPW, FEAT), lambda i: (i, 0)),
          pl.BlockSpec((BPW, FEAT), lambda i: (i, 0)),
          pl.BlockSpec((3, FEAT, OUT_DIM), lambda i: (0, 0, 0)),
          pl.BlockSpec((8, OUT_DIM), lambda i: (0, 0)),
      ],
      out_specs=pl.BlockSpec((BPW, OUT_DIM), lambda i: (i, 0)),
      out_shape=jax.ShapeDtypeStruct((B, OUT_DIM), jnp.float32),
  )(ty3, g0, g1, g2, wp, bp)


def kernel(batch, node_types, feat_map, ip_feats, domain_feats, url_feats,
           W_ip, b_ip, W_dom, b_dom, W_url, b_url):
  batch = batch.astype(jnp.int32)
  node_types = node_types.astype(jnp.int32)
  feat_map = feat_map.astype(jnp.int32)

  ty_b, g0, g1, g2 = _sc_gather()(batch, node_types, feat_map,
                                  ip_feats, domain_feats, url_feats)

  d = OUT_DIM - NUM_NODE_TYPES
  wp = jnp.zeros((3, FEAT, OUT_DIM), jnp.float32)
  bp = jnp.zeros((8, OUT_DIM), jnp.float32)
  for t, (w, b) in enumerate(((W_ip, b_ip), (W_dom, b_dom), (W_url, b_url))):
    wp = wp.at[t, :, NUM_NODE_TYPES:].set(w.T.astype(jnp.float32))
    bp = bp.at[t, NUM_NODE_TYPES:].set(b.astype(jnp.float32))

  ty_col = ty_b.reshape(B, 1)
  return _tc_project(ty_col, g0, g1, g2, wp, bp)


# speedup vs baseline: 2.6873x; 1.0060x over previous
import functools

import jax
import jax.numpy as jnp
from jax import lax
from jax.experimental import pallas as pl
from jax.experimental.pallas import tpu as pltpu
from jax.experimental.pallas import tpu_sc as plsc

OUT_DIM = 64
NUM_NODE_TYPES = 5
FEAT = 128
B = 16384
NUM_CORES = 2
NUM_SUBCORES = 16
NW = NUM_CORES * NUM_SUBCORES
BPW = B // NW


_NBUF = 4
_Q = BPW // _NBUF


def _sc_gather_body(batch_hbm, nt_hbm, fm_hbm, t0_hbm, t1_hbm, t2_hbm,
                    ty_out, g0_out, g1_out, g2_out,
                    bidx_v, ty_v, fi_v, rows_v,
                    sem_nt, sem_fm, sem_ty,
                    gs0, gs1, gs2, gs3, ws0, ws1, ws2, ws3):
  gsems = (gs0, gs1, gs2, gs3)
  wsems = (ws0, ws1, ws2, ws3)
  wid = lax.axis_index("s") * NUM_CORES + lax.axis_index("c")
  base = wid * BPW
  pltpu.sync_copy(batch_hbm.at[pl.ds(base, BPW)], bidx_v)
  h_nt = pltpu.async_copy(nt_hbm.at[bidx_v], ty_v, sem_nt)
  h_fm = pltpu.async_copy(fm_hbm.at[bidx_v], fi_v, sem_fm)
  h_nt.wait()
  h_ty = pltpu.async_copy(ty_v, ty_out.at[pl.ds(base, BPW)], sem_ty)
  h_fm.wait()

  tables = (t0_hbm, t1_hbm, t2_hbm)
  outs = (g0_out, g1_out, g2_out)
  nchunk = 3 * _NBUF

  def start_gather(k):
    t, q = k // _NBUF, k % _NBUF
    return pltpu.async_copy(
        tables[t].at[fi_v.at[pl.ds(q * _Q, _Q)]],
        rows_v.at[pl.ds((k % _NBUF) * _Q, _Q)], gsems[k % _NBUF])

  gh = [None] * nchunk
  wh = [None] * nchunk
  gh[0] = start_gather(0)
  for k in range(nchunk):
    if k + 1 < nchunk:
      if k + 1 >= _NBUF:
        wh[k + 1 - _NBUF].wait()
      gh[k + 1] = start_gather(k + 1)
    gh[k].wait()
    t, q = k // _NBUF, k % _NBUF
    wh[k] = pltpu.async_copy(
        rows_v.at[pl.ds((k % _NBUF) * _Q, _Q)],
        outs[t].at[pl.ds(base + q * _Q, _Q)], wsems[k % _NBUF])
  for k in range(nchunk - _NBUF, nchunk):
    wh[k].wait()
  h_ty.wait()


@functools.cache
def _sc_gather():
  return pl.kernel(
      _sc_gather_body,
      out_type=[
          jax.ShapeDtypeStruct((B,), jnp.int32),
          jax.ShapeDtypeStruct((B, FEAT), jnp.float32),
          jax.ShapeDtypeStruct((B, FEAT), jnp.float32),
          jax.ShapeDtypeStruct((B, FEAT), jnp.float32),
      ],
      mesh=plsc.VectorSubcoreMesh(core_axis_name="c", subcore_axis_name="s"),
      scratch_types=[
          pltpu.VMEM((BPW,), jnp.int32),
          pltpu.VMEM((BPW,), jnp.int32),
          pltpu.VMEM((BPW,), jnp.int32),
          pltpu.VMEM((BPW, FEAT), jnp.float32),
      ] + [pltpu.SemaphoreType.DMA] * 11,
  )


def _tc_project_body(ty_ref, g0_ref, g1_ref, g2_ref, w_ref, b_ref, o_ref):
  ty = ty_ref[...]
  cols = lax.broadcasted_iota(jnp.int32, (BPW, OUT_DIM), 1)
  acc = (cols == ty).astype(jnp.float32)
  for t, g_ref in enumerate((g0_ref, g1_ref, g2_ref)):
    p = jnp.dot(g_ref[...], w_ref[t], preferred_element_type=jnp.float32,
                precision=lax.Precision.HIGHEST)
    p = p + b_ref[t][None, :]
    acc = acc + jnp.where(ty == t, p, 0.0)
  o_ref[...] = acc


def _tc_project(ty3, g0, g1, g2, wp, bp):
  return pl.pallas_call(
      _tc_project_body,
      grid=(NW,),
      in_specs=[
          pl.BlockSpec((BPW, 1), lambda i: (i, 0)),
          pl.BlockSpec((BPW, FEAT), lambda i: (i, 0)),
          pl.BlockSpec((BPW, FEAT), lambda i: (i, 0)),
          pl.BlockSpec((BPW, FEAT), lambda i: (i, 0)),
          pl.BlockSpec((3, FEAT, OUT_DIM), lambda i: (0, 0, 0)),
          pl.BlockSpec((8, OUT_DIM), lambda i: (0, 0)),
      ],
      out_specs=pl.BlockSpec((BPW, OUT_DIM), lambda i: (i, 0)),
      out_shape=jax.ShapeDtypeStruct((B, OUT_DIM), jnp.float32),
  )(ty3, g0, g1, g2, wp, bp)


def kernel(batch, node_types, feat_map, ip_feats, domain_feats, url_feats,
           W_ip, b_ip, W_dom, b_dom, W_url, b_url):
  batch = batch.astype(jnp.int32)
  node_types = node_types.astype(jnp.int32)
  feat_map = feat_map.astype(jnp.int32)

  ty_b, g0, g1, g2 = _sc_gather()(batch, node_types, feat_map,
                                  ip_feats, domain_feats, url_feats)

  d = OUT_DIM - NUM_NODE_TYPES
  wp = jnp.zeros((3, FEAT, OUT_DIM), jnp.float32)
  bp = jnp.zeros((8, OUT_DIM), jnp.float32)
  for t, (w, b) in enumerate(((W_ip, b_ip), (W_dom, b_dom), (W_url, b_url))):
    wp = wp.at[t, :, NUM_NODE_TYPES:].set(w.T.astype(jnp.float32))
    bp = bp.at[t, NUM_NODE_TYPES:].set(b.astype(jnp.float32))

  ty_col = ty_b.reshape(B, 1)
  return _tc_project(ty_col, g0, g1, g2, wp, bp)


# speedup vs baseline: 2.9048x; 1.0809x over previous
import functools

import jax
import jax.numpy as jnp
from jax import lax
from jax.experimental import pallas as pl
from jax.experimental.pallas import tpu as pltpu
from jax.experimental.pallas import tpu_sc as plsc

OUT_DIM = 64
NUM_NODE_TYPES = 5
FEAT = 128
B = 16384
NUM_CORES = 2
NUM_SUBCORES = 16
NW = NUM_CORES * NUM_SUBCORES
BPW = B // NW


_NBUF = 4
_Q = BPW // _NBUF


def _sc_gather_body(batch_hbm, nt_hbm, fm_hbm, t0_hbm, t1_hbm, t2_hbm,
                    ty_out, g0_out, g1_out, g2_out,
                    bidx_v, ty_v, fi_v, rows_v,
                    sem_nt, sem_fm, sem_ty,
                    gs0, gs1, gs2, gs3, ws0, ws1, ws2, ws3):
  gsems = (gs0, gs1, gs2, gs3)
  wsems = (ws0, ws1, ws2, ws3)
  wid = lax.axis_index("s") * NUM_CORES + lax.axis_index("c")
  base = wid * BPW
  pltpu.sync_copy(batch_hbm.at[pl.ds(base, BPW)], bidx_v)
  h_nt = pltpu.async_copy(nt_hbm.at[bidx_v], ty_v, sem_nt)
  h_fm = pltpu.async_copy(fm_hbm.at[bidx_v], fi_v, sem_fm)
  h_nt.wait()
  h_ty = pltpu.async_copy(ty_v, ty_out.at[pl.ds(base, BPW)], sem_ty)
  h_fm.wait()

  tables = (t0_hbm, t1_hbm, t2_hbm)
  outs = (g0_out, g1_out, g2_out)
  nchunk = 3 * _NBUF

  def start_gather(k):
    t, q = k // _NBUF, k % _NBUF
    return pltpu.async_copy(
        tables[t].at[fi_v.at[pl.ds(q * _Q, _Q)]],
        rows_v.at[pl.ds((k % _NBUF) * _Q, _Q)], gsems[k % _NBUF])

  gh = [None] * nchunk
  wh = [None] * nchunk
  gh[0] = start_gather(0)
  for k in range(nchunk):
    if k + 1 < nchunk:
      if k + 1 >= _NBUF:
        wh[k + 1 - _NBUF].wait()
      gh[k + 1] = start_gather(k + 1)
    gh[k].wait()
    t, q = k // _NBUF, k % _NBUF
    wh[k] = pltpu.async_copy(
        rows_v.at[pl.ds((k % _NBUF) * _Q, _Q)],
        outs[t].at[pl.ds(base + q * _Q, _Q)], wsems[k % _NBUF])
  for k in range(nchunk - _NBUF, nchunk):
    wh[k].wait()
  h_ty.wait()


@functools.cache
def _sc_gather():
  return pl.kernel(
      _sc_gather_body,
      out_type=[
          jax.ShapeDtypeStruct((B,), jnp.int32),
          jax.ShapeDtypeStruct((B, FEAT), jnp.float32),
          jax.ShapeDtypeStruct((B, FEAT), jnp.float32),
          jax.ShapeDtypeStruct((B, FEAT), jnp.float32),
      ],
      mesh=plsc.VectorSubcoreMesh(core_axis_name="c", subcore_axis_name="s"),
      scratch_types=[
          pltpu.VMEM((BPW,), jnp.int32),
          pltpu.VMEM((BPW,), jnp.int32),
          pltpu.VMEM((BPW,), jnp.int32),
          pltpu.VMEM((BPW, FEAT), jnp.float32),
      ] + [pltpu.SemaphoreType.DMA] * 11,
  )


_KBIG = 4 * FEAT


def _tc_project_body(ty_ref, g0_ref, g1_ref, g2_ref, w_ref, o_ref):
  ty = ty_ref[...]
  lanes = lax.broadcasted_iota(jnp.int32, (BPW, FEAT), 1)
  oh = (lanes == ty).astype(jnp.float32)
  parts = [jnp.where(ty == t, g_ref[...], 0.0)
           for t, g_ref in enumerate((g0_ref, g1_ref, g2_ref))]
  parts.append(oh)
  big = jnp.concatenate(parts, axis=1)
  o_ref[...] = jnp.dot(big, w_ref[...], preferred_element_type=jnp.float32)


def _tc_project(ty_col, g0, g1, g2, wb):
  return pl.pallas_call(
      _tc_project_body,
      grid=(NW,),
      in_specs=[
          pl.BlockSpec((BPW, 1), lambda i: (i, 0)),
          pl.BlockSpec((BPW, FEAT), lambda i: (i, 0)),
          pl.BlockSpec((BPW, FEAT), lambda i: (i, 0)),
          pl.BlockSpec((BPW, FEAT), lambda i: (i, 0)),
          pl.BlockSpec((_KBIG, OUT_DIM), lambda i: (0, 0)),
      ],
      out_specs=pl.BlockSpec((BPW, OUT_DIM), lambda i: (i, 0)),
      out_shape=jax.ShapeDtypeStruct((B, OUT_DIM), jnp.float32),
  )(ty_col, g0, g1, g2, wb)


def kernel(batch, node_types, feat_map, ip_feats, domain_feats, url_feats,
           W_ip, b_ip, W_dom, b_dom, W_url, b_url):
  batch = batch.astype(jnp.int32)
  node_types = node_types.astype(jnp.int32)
  feat_map = feat_map.astype(jnp.int32)

  ty_b, g0, g1, g2 = _sc_gather()(batch, node_types, feat_map,
                                  ip_feats, domain_feats, url_feats)

  top = jnp.concatenate(
      [W_ip.T, W_dom.T, W_url.T], axis=0).astype(jnp.float32)
  top = jnp.pad(top, ((0, 0), (NUM_NODE_TYPES, 0)))
  bias3 = jnp.stack([b_ip, b_dom, b_url]).astype(jnp.float32)
  bias5 = jnp.pad(bias3, ((0, 2), (0, 0)))
  mid = jnp.concatenate([jnp.eye(NUM_NODE_TYPES, dtype=jnp.float32),
                         bias5], axis=1)
  wb = jnp.concatenate(
      [top, mid,
       jnp.zeros((_KBIG - 384 - NUM_NODE_TYPES, OUT_DIM), jnp.float32)],
      axis=0)

  ty_col = ty_b.reshape(B, 1)
  return _tc_project(ty_col, g0, g1, g2, wb)


# speedup vs baseline: 3.3716x; 1.1607x over previous
import functools

import jax
import jax.numpy as jnp
from jax import lax
from jax.experimental import pallas as pl
from jax.experimental.pallas import tpu as pltpu
from jax.experimental.pallas import tpu_sc as plsc

OUT_DIM = 64
NUM_NODE_TYPES = 5
FEAT = 128
B = 16384
NUM_CORES = 2
NUM_SUBCORES = 16
NW = NUM_CORES * NUM_SUBCORES
BPW = B // NW


_NBUF = 4
_Q = BPW // _NBUF


def _sc_gather_body(batch_hbm, nt_hbm, fm_hbm, t0_hbm, t1_hbm, t2_hbm,
                    ty_out, g0_out, g1_out, g2_out,
                    bidx_v, ty_v, fi_v, rows_v,
                    sem_nt, sem_fm, sem_ty,
                    gs0, gs1, gs2, gs3, ws0, ws1, ws2, ws3):
  gsems = (gs0, gs1, gs2, gs3)
  wsems = (ws0, ws1, ws2, ws3)
  wid = lax.axis_index("s") * NUM_CORES + lax.axis_index("c")
  base = wid * BPW
  pltpu.sync_copy(batch_hbm.at[pl.ds(base, BPW)], bidx_v)
  h_nt = pltpu.async_copy(nt_hbm.at[bidx_v], ty_v, sem_nt)
  h_fm = pltpu.async_copy(fm_hbm.at[bidx_v], fi_v, sem_fm)
  h_nt.wait()
  h_ty = pltpu.async_copy(ty_v, ty_out.at[pl.ds(base, BPW)], sem_ty)
  h_fm.wait()

  tables = (t0_hbm, t1_hbm, t2_hbm)
  outs = (g0_out, g1_out, g2_out)
  nchunk = 3 * _NBUF

  def start_gather(k):
    t, q = k // _NBUF, k % _NBUF
    return pltpu.async_copy(
        tables[t].at[fi_v.at[pl.ds(q * _Q, _Q)]],
        rows_v.at[pl.ds((k % _NBUF) * _Q, _Q)], gsems[k % _NBUF])

  gh = [None] * nchunk
  wh = [None] * nchunk
  gh[0] = start_gather(0)
  for k in range(nchunk):
    if k + 1 < nchunk:
      if k + 1 >= _NBUF:
        wh[k + 1 - _NBUF].wait()
      gh[k + 1] = start_gather(k + 1)
    gh[k].wait()
    t, q = k // _NBUF, k % _NBUF
    wh[k] = pltpu.async_copy(
        rows_v.at[pl.ds((k % _NBUF) * _Q, _Q)],
        outs[t].at[pl.ds(base + q * _Q, _Q)], wsems[k % _NBUF])
  for k in range(nchunk - _NBUF, nchunk):
    wh[k].wait()
  h_ty.wait()


@functools.cache
def _sc_gather():
  return pl.kernel(
      _sc_gather_body,
      out_type=[
          jax.ShapeDtypeStruct((B,), jnp.int32),
          jax.ShapeDtypeStruct((B, FEAT), jnp.float32),
          jax.ShapeDtypeStruct((B, FEAT), jnp.float32),
          jax.ShapeDtypeStruct((B, FEAT), jnp.float32),
      ],
      mesh=plsc.VectorSubcoreMesh(core_axis_name="c", subcore_axis_name="s"),
      scratch_types=[
          pltpu.VMEM((BPW,), jnp.int32),
          pltpu.VMEM((BPW,), jnp.int32),
          pltpu.VMEM((BPW,), jnp.int32),
          pltpu.VMEM((BPW, FEAT), jnp.float32),
      ] + [pltpu.SemaphoreType.DMA] * 11,
  )


_KBIG = 4 * FEAT
_BTC = 2048
_GRID = B // _BTC


def _tc_project_body(ty_ref, g0_ref, g1_ref, g2_ref, w_ref, o_ref):
  ty = ty_ref[...]
  lanes = lax.broadcasted_iota(jnp.int32, (_BTC, FEAT), 1)
  oh = (lanes == ty).astype(jnp.float32)
  parts = [jnp.where(ty == t, g_ref[...], 0.0)
           for t, g_ref in enumerate((g0_ref, g1_ref, g2_ref))]
  parts.append(oh)
  big = jnp.concatenate(parts, axis=1)
  o_ref[...] = jnp.dot(big, w_ref[...], preferred_element_type=jnp.float32)


def _tc_project(ty_col, g0, g1, g2, wb):
  return pl.pallas_call(
      _tc_project_body,
      grid=(_GRID,),
      in_specs=[
          pl.BlockSpec((_BTC, 1), lambda i: (i, 0)),
          pl.BlockSpec((_BTC, FEAT), lambda i: (i, 0)),
          pl.BlockSpec((_BTC, FEAT), lambda i: (i, 0)),
          pl.BlockSpec((_BTC, FEAT), lambda i: (i, 0)),
          pl.BlockSpec((_KBIG, OUT_DIM), lambda i: (0, 0)),
      ],
      out_specs=pl.BlockSpec((_BTC, OUT_DIM), lambda i: (i, 0)),
      out_shape=jax.ShapeDtypeStruct((B, OUT_DIM), jnp.float32),
  )(ty_col, g0, g1, g2, wb)


def kernel(batch, node_types, feat_map, ip_feats, domain_feats, url_feats,
           W_ip, b_ip, W_dom, b_dom, W_url, b_url):
  batch = batch.astype(jnp.int32)
  node_types = node_types.astype(jnp.int32)
  feat_map = feat_map.astype(jnp.int32)

  ty_b, g0, g1, g2 = _sc_gather()(batch, node_types, feat_map,
                                  ip_feats, domain_feats, url_feats)

  top = jnp.concatenate(
      [W_ip.T, W_dom.T, W_url.T], axis=0).astype(jnp.float32)
  top = jnp.pad(top, ((0, 0), (NUM_NODE_TYPES, 0)))
  bias3 = jnp.stack([b_ip, b_dom, b_url]).astype(jnp.float32)
  bias5 = jnp.pad(bias3, ((0, 2), (0, 0)))
  mid = jnp.concatenate([jnp.eye(NUM_NODE_TYPES, dtype=jnp.float32),
                         bias5], axis=1)
  wb = jnp.concatenate(
      [top, mid,
       jnp.zeros((_KBIG - 384 - NUM_NODE_TYPES, OUT_DIM), jnp.float32)],
      axis=0)

  ty_col = ty_b.reshape(B, 1)
  return _tc_project(ty_col, g0, g1, g2, wb)


# speedup vs baseline: 3.4224x; 1.0150x over previous
import functools

import jax
import jax.numpy as jnp
from jax import lax
from jax.experimental import pallas as pl
from jax.experimental.pallas import tpu as pltpu
from jax.experimental.pallas import tpu_sc as plsc

OUT_DIM = 64
NUM_NODE_TYPES = 5
FEAT = 128
B = 16384
NUM_CORES = 2
NUM_SUBCORES = 16
NW = NUM_CORES * NUM_SUBCORES
BPW = B // NW


_NBUF = 4
_Q = BPW // _NBUF


def _sc_gather_body(batch_hbm, nt_hbm, fm_hbm, t0_hbm, t1_hbm, t2_hbm,
                    ty_out, g0_out, g1_out, g2_out,
                    bidx_v, ty_v, fi_v, rows_v,
                    sem_nt, sem_fm, sem_ty,
                    gs0, gs1, gs2, gs3, ws0, ws1, ws2, ws3):
  gsems = (gs0, gs1, gs2, gs3)
  wsems = (ws0, ws1, ws2, ws3)
  wid = lax.axis_index("s") * NUM_CORES + lax.axis_index("c")
  base = wid * BPW
  pltpu.sync_copy(batch_hbm.at[pl.ds(base, BPW)], bidx_v)
  h_nt = pltpu.async_copy(nt_hbm.at[bidx_v], ty_v, sem_nt)
  h_fm = pltpu.async_copy(fm_hbm.at[bidx_v], fi_v, sem_fm)
  h_nt.wait()
  h_ty = pltpu.async_copy(ty_v, ty_out.at[pl.ds(base, BPW)], sem_ty)
  h_fm.wait()

  tables = (t0_hbm, t1_hbm, t2_hbm)
  outs = (g0_out, g1_out, g2_out)
  nchunk = 3 * _NBUF

  def start_gather(k):
    t, q = k // _NBUF, k % _NBUF
    return pltpu.async_copy(
        tables[t].at[fi_v.at[pl.ds(q * _Q, _Q)]],
        rows_v.at[pl.ds((k % _NBUF) * _Q, _Q)], gsems[k % _NBUF])

  gh = [None] * nchunk
  wh = [None] * nchunk
  gh[0] = start_gather(0)
  for k in range(nchunk):
    if k + 1 < nchunk:
      if k + 1 >= _NBUF:
        wh[k + 1 - _NBUF].wait()
      gh[k + 1] = start_gather(k + 1)
    gh[k].wait()
    t, q = k // _NBUF, k % _NBUF
    wh[k] = pltpu.async_copy(
        rows_v.at[pl.ds((k % _NBUF) * _Q, _Q)],
        outs[t].at[pl.ds(base + q * _Q, _Q)], wsems[k % _NBUF])
  for k in range(nchunk - _NBUF, nchunk):
    wh[k].wait()
  h_ty.wait()


@functools.cache
def _sc_gather():
  return pl.kernel(
      _sc_gather_body,
      out_type=[
          jax.ShapeDtypeStruct((B,), jnp.int32),
          jax.ShapeDtypeStruct((B, FEAT), jnp.float32),
          jax.ShapeDtypeStruct((B, FEAT), jnp.float32),
          jax.ShapeDtypeStruct((B, FEAT), jnp.float32),
      ],
      mesh=plsc.VectorSubcoreMesh(core_axis_name="c", subcore_axis_name="s"),
      scratch_types=[
          pltpu.VMEM((BPW,), jnp.int32),
          pltpu.VMEM((BPW,), jnp.int32),
          pltpu.VMEM((BPW,), jnp.int32),
          pltpu.VMEM((BPW, FEAT), jnp.float32),
      ] + [pltpu.SemaphoreType.DMA] * 11,
  )


_KBIG = 4 * FEAT
_BTC = 4096
_GRID = B // _BTC


def _tc_project_body(ty_ref, g0_ref, g1_ref, g2_ref, w_ref, o_ref):
  ty = ty_ref[...]
  lanes = lax.broadcasted_iota(jnp.int32, (_BTC, FEAT), 1)
  oh = (lanes == ty).astype(jnp.float32)
  parts = [jnp.where(ty == t, g_ref[...], 0.0)
           for t, g_ref in enumerate((g0_ref, g1_ref, g2_ref))]
  parts.append(oh)
  big = jnp.concatenate(parts, axis=1)
  o_ref[...] = jnp.dot(big, w_ref[...], preferred_element_type=jnp.float32)


def _tc_project(ty_col, g0, g1, g2, wb):
  return pl.pallas_call(
      _tc_project_body,
      grid=(_GRID,),
      in_specs=[
          pl.BlockSpec((_BTC, 1), lambda i: (i, 0)),
          pl.BlockSpec((_BTC, FEAT), lambda i: (i, 0)),
          pl.BlockSpec((_BTC, FEAT), lambda i: (i, 0)),
          pl.BlockSpec((_BTC, FEAT), lambda i: (i, 0)),
          pl.BlockSpec((_KBIG, OUT_DIM), lambda i: (0, 0)),
      ],
      out_specs=pl.BlockSpec((_BTC, OUT_DIM), lambda i: (i, 0)),
      out_shape=jax.ShapeDtypeStruct((B, OUT_DIM), jnp.float32),
  )(ty_col, g0, g1, g2, wb)


def kernel(batch, node_types, feat_map, ip_feats, domain_feats, url_feats,
           W_ip, b_ip, W_dom, b_dom, W_url, b_url):
  batch = batch.astype(jnp.int32)
  node_types = node_types.astype(jnp.int32)
  feat_map = feat_map.astype(jnp.int32)

  ty_b, g0, g1, g2 = _sc_gather()(batch, node_types, feat_map,
                                  ip_feats, domain_feats, url_feats)

  top = jnp.concatenate(
      [W_ip.T, W_dom.T, W_url.T], axis=0).astype(jnp.float32)
  top = jnp.pad(top, ((0, 0), (NUM_NODE_TYPES, 0)))
  bias3 = jnp.stack([b_ip, b_dom, b_url]).astype(jnp.float32)
  bias5 = jnp.pad(bias3, ((0, 2), (0, 0)))
  mid = jnp.concatenate([jnp.eye(NUM_NODE_TYPES, dtype=jnp.float32),
                         bias5], axis=1)
  wb = jnp.concatenate(
      [top, mid,
       jnp.zeros((_KBIG - 384 - NUM_NODE_TYPES, OUT_DIM), jnp.float32)],
      axis=0)

  ty_col = ty_b.reshape(B, 1)
  return _tc_project(ty_col, g0, g1, g2, wb)


# speedup vs baseline: 3.5898x; 1.0489x over previous
import functools

import jax
import jax.numpy as jnp
from jax import lax
from jax.experimental import pallas as pl
from jax.experimental.pallas import tpu as pltpu
from jax.experimental.pallas import tpu_sc as plsc

OUT_DIM = 64
NUM_NODE_TYPES = 5
FEAT = 128
B = 16384
NUM_CORES = 2
NUM_SUBCORES = 16
NW = NUM_CORES * NUM_SUBCORES
BPW = B // NW

_NBUF = 4
_Q = BPW // _NBUF
_NT = 3


def _sc_gather_body(batch_hbm, nt_hbm, fm_hbm, t0_hbm, t1_hbm, t2_hbm,
                    ty_out, g_out,
                    bidx_v, ty_v, fi_v, rows_v,
                    mi0, mi1, mi2, ps0, ps1, ps2,
                    sem_nt, sem_fm, sem_ty,
                    gs0, gs1, gs2, gs3, ws0, ws1, ws2, ws3):
  gsems = (gs0, gs1, gs2, gs3)
  wsems = (ws0, ws1, ws2, ws3)
  midx = (mi0, mi1, mi2)
  pos2 = (ps0, ps1, ps2)
  wid = lax.axis_index("s") * NUM_CORES + lax.axis_index("c")
  base = wid * BPW
  pltpu.sync_copy(batch_hbm.at[pl.ds(base, BPW)], bidx_v)
  h_nt = pltpu.async_copy(nt_hbm.at[bidx_v], ty_v, sem_nt)
  h_fm = pltpu.async_copy(fm_hbm.at[bidx_v], fi_v, sem_fm)
  h_nt.wait()
  h_ty = pltpu.async_copy(ty_v, ty_out.at[pl.ds(base, BPW)], sem_ty)
  h_fm.wait()

  lane = lax.iota(jnp.int32, 16)
  for j in range(BPW // 16):
    tyv = ty_v[pl.ds(j * 16, 16)]
    fiv = fi_v[pl.ds(j * 16, 16)]
    posb = base + j * 16 + lane
    for t in range(_NT):
      m = tyv == t
      midx[t][pl.ds(j * 16, 16)] = fiv
      pos2[t][j // 8, pl.ds((j % 8) * 16, 16)] = jnp.where(m, posb, posb + B)

  tables = (t0_hbm, t1_hbm, t2_hbm)
  nchunk = _NT * _NBUF

  def start_gather(k):
    t, q = k // _NBUF, k % _NBUF
    return pltpu.async_copy(
        tables[t].at[midx[t].at[pl.ds(q * _Q, _Q)]],
        rows_v.at[pl.ds((k % _NBUF) * _Q, _Q)], gsems[k % _NBUF])

  gh = [None] * nchunk
  wh = [None] * nchunk
  gh[0] = start_gather(0)
  for k in range(nchunk):
    if k + 1 < nchunk:
      if k + 1 >= _NBUF:
        wh[k + 1 - _NBUF].wait()
      gh[k + 1] = start_gather(k + 1)
    gh[k].wait()
    t, q = k // _NBUF, k % _NBUF
    wh[k] = pltpu.async_copy(
        rows_v.at[pl.ds((k % _NBUF) * _Q, _Q)],
        g_out.at[pos2[t].at[q]], wsems[k % _NBUF])
  for k in range(nchunk - _NBUF, nchunk):
    wh[k].wait()
  h_ty.wait()


@functools.cache
def _sc_gather():
  return pl.kernel(
      _sc_gather_body,
      out_type=[
          jax.ShapeDtypeStruct((B,), jnp.int32),
          jax.ShapeDtypeStruct((2 * B, FEAT), jnp.float32),
      ],
      mesh=plsc.VectorSubcoreMesh(core_axis_name="c", subcore_axis_name="s"),
      scratch_types=[
          pltpu.VMEM((BPW,), jnp.int32),
          pltpu.VMEM((BPW,), jnp.int32),
          pltpu.VMEM((BPW,), jnp.int32),
          pltpu.VMEM((BPW, FEAT), jnp.float32),
          pltpu.VMEM((BPW,), jnp.int32),
          pltpu.VMEM((BPW,), jnp.int32),
          pltpu.VMEM((BPW,), jnp.int32),
          pltpu.VMEM((_NBUF, _Q), jnp.int32),
          pltpu.VMEM((_NBUF, _Q), jnp.int32),
          pltpu.VMEM((_NBUF, _Q), jnp.int32),
      ] + [pltpu.SemaphoreType.DMA] * 11,
  )


_K2 = 2 * FEAT
_N2 = _NT * OUT_DIM
_BTC = 4096
_GRID = B // _BTC


def _tc_project_body(ty_ref, g_ref, w_ref, o_ref):
  ty = ty_ref[...]
  big = jnp.concatenate(
      [g_ref[...], jnp.ones((_BTC, FEAT), jnp.float32)], axis=1)
  p = jnp.dot(big, w_ref[...], preferred_element_type=jnp.float32)
  lanes = lax.broadcasted_iota(jnp.int32, (_BTC, OUT_DIM), 1)
  acc = (lanes == ty).astype(jnp.float32)
  for t in range(_NT):
    acc = acc + jnp.where(ty == t, p[:, t * OUT_DIM:(t + 1) * OUT_DIM], 0.0)
  o_ref[...] = acc


def _tc_project(ty_col, g, wc):
  return pl.pallas_call(
      _tc_project_body,
      grid=(_GRID,),
      in_specs=[
          pl.BlockSpec((_BTC, 1), lambda i: (i, 0)),
          pl.BlockSpec((_BTC, FEAT), lambda i: (i, 0)),
          pl.BlockSpec((_K2, _N2), lambda i: (0, 0)),
      ],
      out_specs=pl.BlockSpec((_BTC, OUT_DIM), lambda i: (i, 0)),
      out_shape=jax.ShapeDtypeStruct((B, OUT_DIM), jnp.float32),
  )(ty_col, g, wc)


def kernel(batch, node_types, feat_map, ip_feats, domain_feats, url_feats,
           W_ip, b_ip, W_dom, b_dom, W_url, b_url):
  batch = batch.astype(jnp.int32)
  node_types = node_types.astype(jnp.int32)
  feat_map = feat_map.astype(jnp.int32)

  ty_b, g = _sc_gather()(batch, node_types, feat_map,
                         ip_feats, domain_feats, url_feats)

  blocks = []
  for w, b in ((W_ip, b_ip), (W_dom, b_dom), (W_url, b_url)):
    top = jnp.pad(w.T.astype(jnp.float32), ((0, 0), (NUM_NODE_TYPES, 0)))
    brow = jnp.pad(b.astype(jnp.float32)[None, :],
                   ((0, 0), (NUM_NODE_TYPES, 0)))
    blocks.append(jnp.concatenate(
        [top, brow, jnp.zeros((FEAT - 1, OUT_DIM), jnp.float32)], axis=0))
  wc = jnp.concatenate(blocks, axis=1)

  ty_col = ty_b.reshape(B, 1)
  return _tc_project(ty_col, g, wc)
